# Initial kernel scaffold; baseline (speedup 1.0000x reference)
import jax
import jax.numpy as jnp
from jax.experimental import pallas as pl


def kernel(table, obj_idx):
    raise NotImplementedError("write your pallas kernel here")


# speedup vs baseline: 1.7897x; 1.7897x over previous
import functools

import jax
import jax.numpy as jnp
from jax import lax
from jax.experimental import pallas as pl
from jax.experimental.pallas import tpu as pltpu
from jax.experimental.pallas import tpu_sc as plsc

_IN_CH = 128
_OUT_CH = 128
_BATCH = 4096
_W = _IN_CH * _OUT_CH
_ROW = _W + _OUT_CH

_NC = 2
_NS = 16
_NW = _NC * _NS
_BPW = _BATCH // _NW
_CB = 4
_NCHUNK = _BPW // _CB


@jax.jit
def _gather_rows(table, idx):
    mesh = plsc.VectorSubcoreMesh(core_axis_name="c", subcore_axis_name="s")

    @functools.partial(
        pl.kernel,
        mesh=mesh,
        out_type=(
            jax.ShapeDtypeStruct((_BATCH, _W), jnp.float32),
            jax.ShapeDtypeStruct((_BATCH, _OUT_CH), jnp.float32),
        ),
        scratch_types=[
            pltpu.VMEM((_NCHUNK, _CB), jnp.int32),
            pltpu.VMEM((_CB, _ROW), jnp.float32),
            pltpu.SemaphoreType.DMA,
        ],
    )
    def k(table_hbm, idx_hbm, w_hbm, b_hbm, idx_v, rows_v, sem):
        wid = lax.axis_index("s") * _NC + lax.axis_index("c")
        base = wid * _BPW
        pltpu.sync_copy(idx_hbm.at[wid], idx_v)

        def body(c, carry):
            r = base + c * _CB
            pltpu.async_copy(table_hbm.at[idx_v.at[c]], rows_v, sem).wait()
            pltpu.sync_copy(rows_v.at[:, pl.ds(0, _W)], w_hbm.at[pl.ds(r, _CB)])
            pltpu.sync_copy(rows_v.at[:, pl.ds(_W, _OUT_CH)],
                            b_hbm.at[pl.ds(r, _CB)])
            return carry

        lax.fori_loop(0, _NCHUNK, body, 0)

    return k(table, idx)


def kernel(table, obj_idx):
    idx = obj_idx.astype(jnp.int32).reshape(_NW, _NCHUNK, _CB)
    w_flat, b_flat = _gather_rows(table, idx)
    weights = w_flat.reshape(_BATCH, _OUT_CH, _IN_CH)
    biases = b_flat.reshape(_BATCH, 1, _OUT_CH)
    return weights, biases


# speedup vs baseline: 1.8472x; 1.0321x over previous
import functools

import jax
import jax.numpy as jnp
from jax import lax
from jax.experimental import pallas as pl
from jax.experimental.pallas import tpu as pltpu
from jax.experimental.pallas import tpu_sc as plsc

_IN_CH = 128
_OUT_CH = 128
_BATCH = 4096
_W = _IN_CH * _OUT_CH
_ROW = _W + _OUT_CH

_NC = 2
_NS = 16
_NW = _NC * _NS
_BPW = _BATCH // _NW
_R = 4
_NGROUP = _BPW // _R


@jax.jit
def _gather_rows(table, idx):
    mesh = plsc.VectorSubcoreMesh(core_axis_name="c", subcore_axis_name="s")

    @functools.partial(
        pl.kernel,
        mesh=mesh,
        out_type=(
            jax.ShapeDtypeStruct((_BATCH, _W), jnp.float32),
            jax.ShapeDtypeStruct((_BATCH, _OUT_CH), jnp.float32),
        ),
        scratch_types=[
            pltpu.VMEM((_BPW, 1), jnp.int32),
            pltpu.VMEM((_R, 1, _ROW), jnp.float32),
            pltpu.SemaphoreType.DMA,
            pltpu.SemaphoreType.DMA,
            pltpu.SemaphoreType.DMA,
            pltpu.SemaphoreType.DMA,
            pltpu.SemaphoreType.DMA,
            pltpu.SemaphoreType.DMA,
            pltpu.SemaphoreType.DMA,
            pltpu.SemaphoreType.DMA,
        ],
    )
    def k(table_hbm, idx_hbm, w_hbm, b_hbm, idx_v, rows_v,
          sg0, sg1, sg2, sg3, so0, so1, so2, so3):
        sg = (sg0, sg1, sg2, sg3)
        so = (so0, so1, so2, so3)
        wid = lax.axis_index("s") * _NC + lax.axis_index("c")
        base = wid * _BPW
        pltpu.sync_copy(idx_hbm.at[wid], idx_v)

        for b in range(_R):
            pltpu.async_copy(table_hbm.at[idx_v.at[b]], rows_v.at[b], sg[b])

        def body(g, carry):
            outs = []
            for b in range(_R):
                c = g * _R + b
                pltpu.make_async_copy(
                    table_hbm.at[pl.ds(0, 1)], rows_v.at[b], sg[b]).wait()
                ow = pltpu.async_copy(
                    rows_v.at[b, :, pl.ds(0, _W)],
                    w_hbm.at[pl.ds(base + c, 1)], so[b])
                ob = pltpu.async_copy(
                    rows_v.at[b, :, pl.ds(_W, _OUT_CH)],
                    b_hbm.at[pl.ds(base + c, 1)], so[b])
                outs.append((ow, ob))
            for b in range(_R):
                ow, ob = outs[b]
                ow.wait()
                ob.wait()

                @pl.when(g < _NGROUP - 1)
                def _():
                    c_next = (g + 1) * _R + b
                    pltpu.async_copy(
                        table_hbm.at[idx_v.at[c_next]], rows_v.at[b], sg[b])

            return carry

        lax.fori_loop(0, _NGROUP, body, 0)

    return k(table, idx)


def kernel(table, obj_idx):
    idx = obj_idx.astype(jnp.int32).reshape(_NW, _BPW, 1)
    w_flat, b_flat = _gather_rows(table, idx)
    weights = w_flat.reshape(_BATCH, _OUT_CH, _IN_CH)
    biases = b_flat.reshape(_BATCH, 1, _OUT_CH)
    return weights, biases


# speedup vs baseline: 2.6505x; 1.4349x over previous
import functools

import jax
import jax.numpy as jnp
from jax import lax
from jax.experimental import pallas as pl
from jax.experimental.pallas import tpu as pltpu
from jax.experimental.pallas import tpu_sc as plsc

_IN_CH = 128
_OUT_CH = 128
_BATCH = 4096
_LANE = 128
_SUB = _IN_CH + 1
_TROWS = 1000 * _SUB

_NC = 2
_NS = 16
_NW = _NC * _NS
_BPW = _BATCH // _NW
_R = 4
_NGROUP = _BPW // _R


@jax.jit
def _gather_rows(table2, idx_all):
    mesh = plsc.VectorSubcoreMesh(core_axis_name="c", subcore_axis_name="s")

    @functools.partial(
        pl.kernel,
        mesh=mesh,
        out_type=(
            jax.ShapeDtypeStruct((_BATCH * _IN_CH, _LANE), jnp.float32),
            jax.ShapeDtypeStruct((_BATCH, _LANE), jnp.float32),
        ),
        scratch_types=[
            pltpu.VMEM((_SUB, _LANE), jnp.int32),
            pltpu.VMEM((_R, _LANE, _LANE), jnp.float32),
            pltpu.SemaphoreType.DMA,
            pltpu.SemaphoreType.DMA,
            pltpu.SemaphoreType.DMA,
            pltpu.SemaphoreType.DMA,
            pltpu.SemaphoreType.DMA,
            pltpu.SemaphoreType.DMA,
            pltpu.SemaphoreType.DMA,
            pltpu.SemaphoreType.DMA,
        ],
    )
    def k(table_hbm, idx_hbm, w_hbm, b_hbm, idx_v, rows_v,
          sg0, sg1, sg2, sg3, so0, so1, so2, so3):
        sg = (sg0, sg1, sg2, sg3)
        so = (so0, so1, so2, so3)
        wid = lax.axis_index("s") * _NC + lax.axis_index("c")
        wbase = wid * _BPW * _IN_CH
        pltpu.sync_copy(idx_hbm.at[wid], idx_v)

        for b in range(_R):
            pltpu.async_copy(table_hbm.at[idx_v.at[b]], rows_v.at[b], sg[b])

        def body(g, carry):
            outs = []
            for b in range(_R):
                c = g * _R + b
                pltpu.make_async_copy(
                    table_hbm.at[pl.ds(0, _LANE)], rows_v.at[b], sg[b]).wait()
                outs.append(pltpu.async_copy(
                    rows_v.at[b],
                    w_hbm.at[pl.ds(wbase + c * _LANE, _LANE)], so[b]))
            for b in range(_R):
                outs[b].wait()

                @pl.when(g < _NGROUP - 1)
                def _():
                    c_next = (g + 1) * _R + b
                    pltpu.async_copy(
                        table_hbm.at[idx_v.at[c_next]], rows_v.at[b], sg[b])

            return carry

        lax.fori_loop(0, _NGROUP, body, 0)

        pltpu.async_copy(table_hbm.at[idx_v.at[_SUB - 1]], rows_v.at[0],
                         sg[0]).wait()
        pltpu.sync_copy(rows_v.at[0], b_hbm.at[pl.ds(wid * _BPW, _BPW)])

    return k(table2, idx_all)


def kernel(table, obj_idx):
    table2 = table.reshape(_TROWS, _LANE)
    oi = obj_idx.astype(jnp.int32).reshape(_NW, _BPW)
    sub = jnp.arange(_IN_CH, dtype=jnp.int32)
    w_ids = oi[:, :, None] * _SUB + sub[None, None, :]
    b_ids = (oi * _SUB + _IN_CH)[:, None, :]
    idx_all = jnp.concatenate([w_ids, b_ids], axis=1)
    w2, b2 = _gather_rows(table2, idx_all)
    weights = w2.reshape(_BATCH, _OUT_CH, _IN_CH)
    biases = b2.reshape(_BATCH, 1, _OUT_CH)
    return weights, biases


# speedup vs baseline: 3.2837x; 1.2389x over previous
import functools

import jax
import jax.numpy as jnp
from jax import lax
from jax.experimental import pallas as pl
from jax.experimental.pallas import tpu as pltpu
from jax.experimental.pallas import tpu_sc as plsc

_IN_CH = 128
_OUT_CH = 128
_BATCH = 4096
_LANE = 128

_NC = 2
_NS = 16
_NW = _NC * _NS
_BPW = _BATCH // _NW
_R = 4
_NGROUP = _IN_CH // _R


@jax.jit
def _lookup(table, idx):
    mesh = plsc.VectorSubcoreMesh(core_axis_name="c", subcore_axis_name="s")

    @functools.partial(
        pl.kernel,
        mesh=mesh,
        out_type=(
            jax.ShapeDtypeStruct((_BATCH, _OUT_CH, _IN_CH), jnp.float32),
            jax.ShapeDtypeStruct((_BATCH, 1, _OUT_CH), jnp.float32),
        ),
        scratch_types=[
            pltpu.VMEM((_BPW,), jnp.int32),
            pltpu.VMEM((_R, _BPW, _LANE), jnp.float32),
            pltpu.SemaphoreType.DMA,
            pltpu.SemaphoreType.DMA,
            pltpu.SemaphoreType.DMA,
            pltpu.SemaphoreType.DMA,
            pltpu.SemaphoreType.DMA,
            pltpu.SemaphoreType.DMA,
            pltpu.SemaphoreType.DMA,
            pltpu.SemaphoreType.DMA,
        ],
    )
    def k(table_hbm, idx_hbm, w_hbm, b_hbm, idx_v, rows_v,
          sg0, sg1, sg2, sg3, so0, so1, so2, so3):
        sg = (sg0, sg1, sg2, sg3)
        so = (so0, so1, so2, so3)
        wid = lax.axis_index("s") * _NC + lax.axis_index("c")
        base = wid * _BPW
        pltpu.sync_copy(idx_hbm.at[wid], idx_v)

        def gather_block(j, slot):
            pltpu.async_copy(
                table_hbm.at[idx_v, pl.ds(j * _LANE, _LANE)],
                rows_v.at[slot], sg[slot])

        for b in range(_R):
            gather_block(b, b)

        def body(g, carry):
            outs = []
            for b in range(_R):
                j = g * _R + b
                pltpu.make_async_copy(
                    table_hbm.at[pl.ds(0, _BPW), pl.ds(0, _LANE)],
                    rows_v.at[b], sg[b]).wait()
                outs.append(pltpu.async_copy(
                    rows_v.at[b], w_hbm.at[pl.ds(base, _BPW), j], so[b]))
            for b in range(_R):
                outs[b].wait()

                @pl.when(g < _NGROUP - 1)
                def _():
                    gather_block((g + 1) * _R + b, b)

            return carry

        lax.fori_loop(0, _NGROUP, body, 0)

        pltpu.async_copy(
            table_hbm.at[idx_v, pl.ds(_IN_CH * _LANE, _LANE)],
            rows_v.at[0], sg[0]).wait()
        pltpu.sync_copy(rows_v.at[0], b_hbm.at[pl.ds(base, _BPW), 0])

    return k(table, idx)


def kernel(table, obj_idx):
    idx = obj_idx.astype(jnp.int32).reshape(_NW, _BPW)
    return _lookup(table, idx)
